# Initial kernel scaffold; baseline (speedup 1.0000x reference)
import jax
import jax.numpy as jnp
from jax.experimental import pallas as pl


def kernel(m_node0, m_edge0, m_W, m_V, r1_node0, r1_node1, r1_node2, r1_node3, r1_node4, r1_node5, r1_node6, r1_node7, r1_node8, r1_edge0, r1_edge1, r1_edge2, r1_W, r1_V, r2_node0, r2_node1, r2_node2, r2_node3, r2_node4, r2_node5, r2_node6, r2_node7, r2_node8, r2_edge0, r2_edge1, r2_edge2, r2_W, r2_V, h_W, h_V, cw, cb, motif_x, motif_edge_index, motif_edge_attr, motif_batch, raw_x_1, raw_edge_index_1, raw_edge_attr_1, raw_batch_1, raw_x_2, raw_edge_index_2, raw_edge_attr_2, raw_batch_2, heter_edge_index):
    raise NotImplementedError("write your pallas kernel here")


# speedup vs baseline: 1.0431x; 1.0431x over previous
import numpy as np
import jax
import jax.numpy as jnp
from jax import lax
from jax.experimental import pallas as pl
from jax.experimental.pallas import tpu as pltpu

_D = 32
_G = 512
_NVOC = 184
_EVOC = 24
_NODE_OFF = (0, 120, 125, 137, 149, 159, 165, 172, 174)
_EDGE_OFF = (0, 15, 21)
_NODE_PAD_IDX = 176
_EDGE_PAD_IDX = 23

_C00 = (((0,), (0,)), ((), ()))


def _branch_kernel(idx_ref, ntab_ref, etab_ref, w_ref, v_ref, o_ref):
    n = idx_ref.shape[1]
    e = n
    g = o_ref.shape[0]
    idx = idx_ref[...]
    w = w_ref[...]
    v = v_ref[...]
    n_layers = v.shape[0]

    iota_nv = lax.broadcasted_iota(jnp.int32, (_NVOC, n), 0)
    nht = (iota_nv == idx[0:1, :]).astype(jnp.float32)
    for i in range(1, 9):
        nht = nht + (iota_nv == idx[i:i + 1, :]).astype(jnp.float32)
    x = lax.dot_general(nht, ntab_ref[...], _C00,
                        preferred_element_type=jnp.float32)

    iota_ev = lax.broadcasted_iota(jnp.int32, (_EVOC, e), 0)
    eht = (iota_ev == idx[9:10, :]).astype(jnp.float32)
    for i in range(10, 12):
        eht = eht + (iota_ev == idx[i:i + 1, :]).astype(jnp.float32)
    ea = lax.dot_general(eht, etab_ref[...], _C00,
                         preferred_element_type=jnp.float32)

    iota_ne = lax.broadcasted_iota(jnp.int32, (n, e), 0)
    ost = (iota_ne == idx[12:13, :]).astype(jnp.float32)
    odt = (iota_ne == idx[13:14, :]).astype(jnp.float32)
    iota_gn = lax.broadcasted_iota(jnp.int32, (g, n), 0)
    pm = (iota_gn == idx[14:15, :]).astype(jnp.float32)
    pinv = 1.0 / jnp.maximum(jnp.sum(pm, axis=1, keepdims=True), 1.0)

    for l in range(n_layers):
        xg = lax.dot_general(ost, x, _C00,
                             preferred_element_type=jnp.float32)
        msg = jnp.maximum(xg + ea, 0.0)
        agg = jnp.dot(odt, msg, preferred_element_type=jnp.float32)
        h = x + agg
        h1 = jnp.maximum(
            jnp.dot(h, w[2 * l], preferred_element_type=jnp.float32)
            + v[l, 0:1], 0.0)
        h2 = jnp.dot(h1, w[2 * l + 1],
                     preferred_element_type=jnp.float32) + v[l, 1:2]
        x = jnp.maximum(h2 * v[l, 2:3] + v[l, 3:4], 0.0)

    o_ref[...] = jnp.dot(pm, x, preferred_element_type=jnp.float32) * pinv


def _cgin_cls_kernel(a_ref, xf_ref, xr_ref, w_ref, v_ref, cw_ref, cb_ref,
                     o_ref):
    agg = jnp.dot(a_ref[...], xf_ref[...],
                  preferred_element_type=jnp.float32)
    h = xr_ref[...] + agg
    w = w_ref[...]
    v = v_ref[...]
    h1 = jnp.maximum(
        jnp.dot(h, w[0], preferred_element_type=jnp.float32) + v[0, 0:1], 0.0)
    h2 = jnp.dot(h1, w[1], preferred_element_type=jnp.float32) + v[0, 1:2]
    xo = jnp.maximum(h2 * v[0, 2:3] + v[0, 3:4], 0.0)
    o_ref[...] = (jnp.dot(xo, cw_ref[...], preferred_element_type=jnp.float32)
                  + cb_ref[...])


def _run_branches(idx_s, ntab_s, etab_s, w_s, v_s):
    n = idx_s.shape[2]
    return pl.pallas_call(
        _branch_kernel,
        out_shape=jax.ShapeDtypeStruct((3, _G, _D), jnp.float32),
        grid=(3,),
        in_specs=[
            pl.BlockSpec((None, 16, n), lambda b: (b, 0, 0)),
            pl.BlockSpec((None, _NVOC, _D), lambda b: (b, 0, 0)),
            pl.BlockSpec((None, _EVOC, _D), lambda b: (b, 0, 0)),
            pl.BlockSpec((None, 4, _D, _D), lambda b: (b, 0, 0, 0)),
            pl.BlockSpec((None, 2, 4, _D), lambda b: (b, 0, 0, 0)),
        ],
        out_specs=pl.BlockSpec((None, _G, _D), lambda b: (b, 0, 0)),
        compiler_params=pltpu.CompilerParams(
            dimension_semantics=("parallel",)),
    )(idx_s, ntab_s, etab_s, w_s, v_s)


def _run_cgin_classifier(x, adj, h_w, h_v, cw, cb):
    nh = x.shape[0]
    blk = nh // 2
    c = cw.shape[1]
    return pl.pallas_call(
        _cgin_cls_kernel,
        out_shape=jax.ShapeDtypeStruct((nh, c), jnp.float32),
        grid=(2,),
        in_specs=[
            pl.BlockSpec((blk, nh), lambda i: (i, 0)),
            pl.BlockSpec((nh, _D), lambda i: (0, 0)),
            pl.BlockSpec((blk, _D), lambda i: (i, 0)),
            pl.BlockSpec((2, _D, _D), lambda i: (0, 0, 0)),
            pl.BlockSpec((1, 4, _D), lambda i: (0, 0, 0)),
            pl.BlockSpec((_D, c), lambda i: (0, 0)),
            pl.BlockSpec((1, c), lambda i: (0, 0)),
        ],
        out_specs=pl.BlockSpec((blk, c), lambda i: (i, 0)),
        compiler_params=pltpu.CompilerParams(
            dimension_semantics=("parallel",)),
    )(adj, x, x, h_w, h_v, cw, cb)


def _branch_idx_rows(node_idx_rows, edge_idx_rows, src, dst, batch, n):
    pad = jnp.zeros((1, n), jnp.int32)
    return jnp.concatenate(
        [node_idx_rows, edge_idx_rows,
         src[None, :].astype(jnp.int32), dst[None, :].astype(jnp.int32),
         batch[None, :].astype(jnp.int32), pad], axis=0)


def kernel(m_node0, m_edge0, m_W, m_V, r1_node0, r1_node1, r1_node2, r1_node3,
           r1_node4, r1_node5, r1_node6, r1_node7, r1_node8, r1_edge0,
           r1_edge1, r1_edge2, r1_W, r1_V, r2_node0, r2_node1, r2_node2,
           r2_node3, r2_node4, r2_node5, r2_node6, r2_node7, r2_node8,
           r2_edge0, r2_edge1, r2_edge2, r2_W, r2_V, h_W, h_V, cw, cb,
           motif_x, motif_edge_index, motif_edge_attr, motif_batch,
           raw_x_1, raw_edge_index_1, raw_edge_attr_1, raw_batch_1,
           raw_x_2, raw_edge_index_2, raw_edge_attr_2, raw_batch_2,
           heter_edge_index):
    n = motif_x.shape[0]
    e = motif_edge_index.shape[1]

    def _pad_rows(t, h):
        return jnp.concatenate(
            [t, jnp.zeros((h - t.shape[0], _D), jnp.float32)], axis=0)

    ntab_m = _pad_rows(m_node0, _NVOC)
    ntab_r1 = _pad_rows(jnp.concatenate(
        [r1_node0, r1_node1, r1_node2, r1_node3, r1_node4, r1_node5,
         r1_node6, r1_node7, r1_node8], axis=0), _NVOC)
    ntab_r2 = _pad_rows(jnp.concatenate(
        [r2_node0, r2_node1, r2_node2, r2_node3, r2_node4, r2_node5,
         r2_node6, r2_node7, r2_node8], axis=0), _NVOC)
    ntab_s = jnp.stack([ntab_m, ntab_r1, ntab_r2])

    etab_m = _pad_rows(m_edge0, _EVOC)
    etab_r1 = _pad_rows(jnp.concatenate([r1_edge0, r1_edge1, r1_edge2],
                                        axis=0), _EVOC)
    etab_r2 = _pad_rows(jnp.concatenate([r2_edge0, r2_edge1, r2_edge2],
                                        axis=0), _EVOC)
    etab_s = jnp.stack([etab_m, etab_r1, etab_r2])

    node_off = jnp.array(_NODE_OFF, jnp.int32)[:, None]
    edge_off = jnp.array(_EDGE_OFF, jnp.int32)[:, None]
    pad_n = jnp.full((8, n), _NODE_PAD_IDX, jnp.int32)
    m_nrows = jnp.concatenate([motif_x[None, :].astype(jnp.int32), pad_n],
                              axis=0)
    r1_nrows = raw_x_1.T.astype(jnp.int32) + node_off
    r2_nrows = raw_x_2.T.astype(jnp.int32) + node_off
    pad_e = jnp.full((2, e), _EDGE_PAD_IDX, jnp.int32)
    m_erows = jnp.concatenate(
        [motif_edge_attr[None, :].astype(jnp.int32), pad_e], axis=0)
    r1_erows = raw_edge_attr_1.T.astype(jnp.int32) + edge_off
    r2_erows = raw_edge_attr_2.T.astype(jnp.int32) + edge_off

    idx_s = jnp.stack([
        _branch_idx_rows(m_nrows, m_erows, motif_edge_index[0],
                         motif_edge_index[1], motif_batch, n),
        _branch_idx_rows(r1_nrows, r1_erows, raw_edge_index_1[0],
                         raw_edge_index_1[1], raw_batch_1, n),
        _branch_idx_rows(r2_nrows, r2_erows, raw_edge_index_2[0],
                         raw_edge_index_2[1], raw_batch_2, n),
    ])

    w_s = jnp.stack([m_W, r1_W, r2_W])
    v_s = jnp.stack([m_V, r1_V, r2_V])

    branch_out = _run_branches(idx_s, ntab_s, etab_s, w_s, v_s)
    node_feature = branch_out.reshape(3 * _G, _D)

    nh = 3 * _G
    src, dst = heter_edge_index[0], heter_edge_index[1]
    adj = jnp.zeros((nh, nh), jnp.float32).at[dst, src].add(
        1.0, mode="drop")

    logits = _run_cgin_classifier(node_feature, adj, h_W, h_V, cw, cb)

    pred1 = logits[_G:2 * _G, 0:1]
    pred2 = logits[2 * _G:3 * _G, 1:3]
    return pred1, pred2
